# baseline (speedup 1.0000x reference)
import functools

import jax
import jax.numpy as jnp
from jax import lax
from jax.experimental import pallas as pl
from jax.experimental.pallas import tpu as pltpu
from jax.experimental.pallas import tpu_sc as plsc

NUM_LAYERS = 3
LANES = 16
NTILES = 16
NCORES = 2
CHUNK = 128
NBUF = 8
BATCH = 64


def _round_up(a: int, m: int) -> int:
    return (a + m - 1) // m * m


def _mm_body(x_ref, w_ref, o_ref, *, n, bm):
    i = pl.program_id(0)
    rows = i * bm + lax.broadcasted_iota(jnp.int32, (bm, 1), 0)
    z = jnp.dot(x_ref[...], w_ref[...], preferred_element_type=jnp.float32)
    o_ref[...] = jnp.where(rows < n, z, 0.0)


def _project(x, wp, npad):
    n, d = x.shape
    bm = npad // 4 if npad % 4096 == 0 else npad // 2
    while npad // bm * bm != npad:
        bm //= 2
    return pl.pallas_call(
        functools.partial(_mm_body, n=n, bm=bm),
        grid=(npad // bm,),
        in_specs=[
            pl.BlockSpec((bm, d), lambda i: (i, 0)),
            pl.BlockSpec((d, LANES), lambda i: (0, 0)),
        ],
        out_specs=pl.BlockSpec((bm, LANES), lambda i: (i, 0)),
        out_shape=jax.ShapeDtypeStruct((npad, LANES), jnp.float32),
    )(x, wp)


def _make_sc_group(active_core, np_g, ep_g):
    nd = len(np_g)
    rpts = [n // NTILES for n in np_g]
    ncts = [e // NTILES // CHUNK for e in ep_g]
    mesh = plsc.VectorSubcoreMesh(core_axis_name="c", subcore_axis_name="s")

    scr = dict(
        pooled=pltpu.VMEM_SHARED((BATCH, LANES), jnp.float32),
        eir0=pltpu.VMEM((2, NBUF, CHUNK), jnp.int32),
        eir1=pltpu.VMEM((2, NBUF, CHUNK), jnp.int32),
        pstage=pltpu.VMEM((BATCH, LANES), jnp.float32),
        isem0=pltpu.SemaphoreType.DMA,
        isem1=pltpu.SemaphoreType.DMA,
        isem0b=pltpu.SemaphoreType.DMA,
        isem1b=pltpu.SemaphoreType.DMA,
    )
    for d in range(nd):
        scr[f"zA{d}"] = pltpu.VMEM_SHARED((np_g[d], LANES), jnp.float32)
        scr[f"zB{d}"] = pltpu.VMEM_SHARED((np_g[d], LANES), jnp.float32)
    for bi in range(NBUF):
        scr[f"rbuf{bi}"] = pltpu.VMEM((CHUNK, LANES), jnp.float32)
    for bi in range(NBUF):
        scr[f"gsem{bi}"] = pltpu.SemaphoreType.DMA
    for bi in range(NBUF):
        scr[f"ssem{bi}"] = pltpu.SemaphoreType.DMA

    @functools.partial(
        pl.kernel,
        out_type=jax.ShapeDtypeStruct((BATCH, LANES), jnp.float32),
        mesh=mesh,
        compiler_params=pltpu.CompilerParams(use_tc_tiling_on_sc=False),
        scratch_types=scr,
    )
    def sc_kernel(*args, **scrk):
        zhs = args[0:nd]
        ehs = args[nd:2 * nd]
        bhs = args[2 * nd:3 * nd]
        out = args[3 * nd]
        pooled = scrk["pooled"]
        eir0, eir1 = scrk["eir0"], scrk["eir1"]
        pstage = scrk["pstage"]
        isem0, isem1 = scrk["isem0"], scrk["isem1"]
        isem0b, isem1b = scrk["isem0b"], scrk["isem1b"]
        zAs = [scrk[f"zA{d}"] for d in range(nd)]
        zBs = [scrk[f"zB{d}"] for d in range(nd)]
        rbufs = [scrk[f"rbuf{bi}"] for bi in range(NBUF)]
        gsems = [scrk[f"gsem{bi}"] for bi in range(NBUF)]
        ssems = [scrk[f"ssem{bi}"] for bi in range(NBUF)]
        c = lax.axis_index("c")
        s = lax.axis_index("s")

        def grouped(n, issue_load, after_load):
            for g0 in range(0, n, NBUF):
                g = min(NBUF, n - g0)
                lds = [issue_load(g0 + i, i) for i in range(g)]
                sds = []
                for i in range(g):
                    lds[i].wait()
                    sds.extend(after_load(g0 + i, i))
                for sd in sds:
                    sd.wait()

        def load_dim(z_hbm, zA, zB, rpt):
            base = s * rpt

            def ld(j, i):
                sl = pl.ds(base + j * CHUNK, CHUNK)
                return pltpu.async_copy(z_hbm.at[sl], rbufs[i], gsems[i])

            def st(j, i):
                sl = pl.ds(base + j * CHUNK, CHUNK)
                return [pltpu.async_copy(rbufs[i], zA.at[sl], ssems[i]),
                        pltpu.async_copy(rbufs[i], zB.at[sl], gsems[i])]

            grouped(rpt // CHUNK, ld, st)

        def copy_rows(src, dst, rpt):
            base = s * rpt

            def ld(j, i):
                sl = pl.ds(base + j * CHUNK, CHUNK)
                return pltpu.async_copy(src.at[sl], rbufs[i], gsems[i])

            def st(j, i):
                sl = pl.ds(base + j * CHUNK, CHUNK)
                return [pltpu.async_copy(rbufs[i], dst.at[sl], ssems[i])]

            grouped(rpt // CHUNK, ld, st)

        def process_quad(zsrc, zdst, eir):
            gds = [
                pltpu.async_copy(zsrc.at[eir.at[0, bi]], rbufs[bi], gsems[bi])
                for bi in range(NBUF)
            ]
            sds = []
            for bi in range(NBUF):
                gds[bi].wait()
                sds.append(pltpu.async_copy(
                    rbufs[bi], zdst.at[eir.at[1, bi]], ssems[bi], add=True))
            for sd in sds:
                sd.wait()

        def edge_pass(zsrc, zdst, e_hbm, nct):
            base = s * nct
            nq = nct // NBUF
            pltpu.sync_copy(e_hbm.at[0, pl.ds(base, NBUF)], eir0.at[0])
            pltpu.sync_copy(e_hbm.at[1, pl.ds(base, NBUF)], eir0.at[1])

            def pair(h, carry):
                q1 = base + (2 * h + 1) * NBUF
                dB0 = pltpu.async_copy(e_hbm.at[0, pl.ds(q1, NBUF)],
                                       eir1.at[0], isem1)
                dB1 = pltpu.async_copy(e_hbm.at[1, pl.ds(q1, NBUF)],
                                       eir1.at[1], isem1b)
                process_quad(zsrc, zdst, eir0)
                dB0.wait()
                dB1.wait()
                dA0 = pltpu.async_copy(e_hbm.at[0, pl.ds(q1 + NBUF, NBUF)],
                                       eir0.at[0], isem0)
                dA1 = pltpu.async_copy(e_hbm.at[1, pl.ds(q1 + NBUF, NBUF)],
                                       eir0.at[1], isem0b)
                process_quad(zsrc, zdst, eir1)
                dA0.wait()
                dA1.wait()
                return carry

            lax.fori_loop(0, nq // 2, pair, 0)
            if nq % 2 == 1:
                process_quad(zsrc, zdst, eir0)

        def pool_dim(zfin, b_hbm, rpt):
            base = s * rpt

            def ld(j, i):
                sl = pl.ds(base + j * CHUNK, CHUNK)
                pltpu.sync_copy(b_hbm.at[sl], eir0.at[0, i])
                return pltpu.async_copy(zfin.at[sl], rbufs[i], gsems[i])

            def st(j, i):
                return [pltpu.async_copy(rbufs[i], pooled.at[eir0.at[0, i]],
                                         ssems[i], add=True)]

            grouped(rpt // CHUNK, ld, st)

        def run_dims(dims):
            for (zh, zA, zB, eh, bh, nct, rpt) in dims:
                load_dim(zh, zA, zB, rpt)

            @pl.when(s == 0)
            def _():
                zv = jnp.zeros((LANES,), jnp.float32)
                for i in range(BATCH):
                    pstage[i, :] = zv
                pltpu.sync_copy(pstage, pooled)

            plsc.subcore_barrier()

            for layer in range(NUM_LAYERS):
                fwd = layer % 2 == 0
                if layer > 0:
                    for (zh, zA, zB, eh, bh, nct, rpt) in dims:
                        copy_rows(zA if fwd else zB, zB if fwd else zA, rpt)
                    plsc.subcore_barrier()
                for (zh, zA, zB, eh, bh, nct, rpt) in dims:
                    edge_pass(zA if fwd else zB, zB if fwd else zA, eh, nct)
                plsc.subcore_barrier()

            for (zh, zA, zB, eh, bh, nct, rpt) in dims:
                pool_dim(zB if NUM_LAYERS % 2 == 1 else zA, bh, rpt)
            plsc.subcore_barrier()

            @pl.when(s == 0)
            def _():
                pltpu.sync_copy(pooled, pstage)
                pltpu.sync_copy(pstage, out)

        dims = [(zhs[d], zAs[d], zBs[d], ehs[d], bhs[d], ncts[d], rpts[d])
                for d in range(nd)]

        @pl.when(c == active_core)
        def _():
            run_dims(dims)

    return sc_kernel


def kernel(x0, x1, x2, edge_index0, edge_index1, edge_index2,
           batch0, batch1, batch2, W, b):
    xs = [x0, x1, x2]
    eis = [edge_index0, edge_index1, edge_index2]
    bs = [batch0, batch1, batch2]
    ns = [x.shape[0] for x in xs]
    npads = [_round_up(n + 1, NTILES * CHUNK) for n in ns]
    epads = [_round_up(ei.shape[1], NTILES * CHUNK * NBUF) for ei in eis]

    wp = jnp.pad(W, ((0, 0), (0, LANES - W.shape[1])))
    zs = [_project(xs[i], wp, npads[i]) for i in range(3)]
    es = []
    for i in range(3):
        alloc = epads[i] + NBUF * CHUNK
        pad = alloc - eis[i].shape[1]
        ep = jnp.pad(eis[i], ((0, 0), (0, pad)), constant_values=ns[i])
        es.append(ep.reshape(2, alloc // CHUNK, CHUNK))
    bpads = [jnp.pad(bs[i], (0, npads[i] - ns[i])) for i in range(3)]

    sc0 = _make_sc_group(0, npads[0:1], epads[0:1])
    sc1 = _make_sc_group(1, npads[1:3], epads[1:3])
    pool0 = sc0(zs[0], es[0], bpads[0])
    pool12 = sc1(zs[1], zs[2], es[1], es[2], bpads[1], bpads[2])
    return (pool0 + pool12)[:, : W.shape[1]] + b


# speedup vs baseline: 1.5892x; 1.5892x over previous
import functools

import jax
import jax.numpy as jnp
from jax import lax
from jax.experimental import pallas as pl
from jax.experimental.pallas import tpu as pltpu
from jax.experimental.pallas import tpu_sc as plsc

NUM_LAYERS = 3
LANES = 16
NTILES = 16
NCORES = 2
CHUNK = 128
NBUF = 8
BATCH = 64


def _round_up(a: int, m: int) -> int:
    return (a + m - 1) // m * m


def _mm_body(x_ref, w_ref, o_ref, *, n, bm):
    i = pl.program_id(0)
    rows = i * bm + lax.broadcasted_iota(jnp.int32, (bm, 1), 0)
    z = jnp.dot(x_ref[...], w_ref[...], preferred_element_type=jnp.float32)
    o_ref[...] = jnp.where(rows < n, z, 0.0)


def _project(x, wp, npad):
    n, d = x.shape
    bm = npad // 4 if npad % 4096 == 0 else npad // 2
    while npad // bm * bm != npad:
        bm //= 2
    return pl.pallas_call(
        functools.partial(_mm_body, n=n, bm=bm),
        grid=(npad // bm,),
        in_specs=[
            pl.BlockSpec((bm, d), lambda i: (i, 0)),
            pl.BlockSpec((d, LANES), lambda i: (0, 0)),
        ],
        out_specs=pl.BlockSpec((bm, LANES), lambda i: (i, 0)),
        out_shape=jax.ShapeDtypeStruct((npad, LANES), jnp.float32),
    )(x, wp)


def _make_sc_kernel(np_g, ep_g, split):
    nd = len(np_g)
    rpts = [n // NTILES for n in np_g]
    ncts = [e // NTILES // CHUNK for e in ep_g]
    mesh = plsc.VectorSubcoreMesh(core_axis_name="c", subcore_axis_name="s")

    scr = dict(
        pooled=pltpu.VMEM_SHARED((BATCH, LANES), jnp.float32),
        eir0=pltpu.VMEM((2, NBUF, CHUNK), jnp.int32),
        eir1=pltpu.VMEM((2, NBUF, CHUNK), jnp.int32),
        pstage=pltpu.VMEM((BATCH, LANES), jnp.float32),
        isem0=pltpu.SemaphoreType.DMA,
        isem1=pltpu.SemaphoreType.DMA,
        isem0b=pltpu.SemaphoreType.DMA,
        isem1b=pltpu.SemaphoreType.DMA,
    )
    for d in range(nd):
        scr[f"zA{d}"] = pltpu.VMEM_SHARED((np_g[d], LANES), jnp.float32)
        scr[f"zB{d}"] = pltpu.VMEM_SHARED((np_g[d], LANES), jnp.float32)
    for bi in range(NBUF):
        scr[f"rbuf{bi}"] = pltpu.VMEM((CHUNK, LANES), jnp.float32)
    for bi in range(NBUF):
        scr[f"gsem{bi}"] = pltpu.SemaphoreType.DMA
    for bi in range(NBUF):
        scr[f"ssem{bi}"] = pltpu.SemaphoreType.DMA

    @functools.partial(
        pl.kernel,
        out_type=jax.ShapeDtypeStruct((NCORES, BATCH, LANES), jnp.float32),
        mesh=mesh,
        compiler_params=pltpu.CompilerParams(use_tc_tiling_on_sc=False),
        scratch_types=scr,
    )
    def sc_kernel(*args, **scrk):
        zhs = args[0:nd]
        ehs = args[nd:2 * nd]
        bhs = args[2 * nd:3 * nd]
        out = args[3 * nd]
        pooled = scrk["pooled"]
        eir0, eir1 = scrk["eir0"], scrk["eir1"]
        pstage = scrk["pstage"]
        isem0, isem1 = scrk["isem0"], scrk["isem1"]
        isem0b, isem1b = scrk["isem0b"], scrk["isem1b"]
        zAs = [scrk[f"zA{d}"] for d in range(nd)]
        zBs = [scrk[f"zB{d}"] for d in range(nd)]
        rbufs = [scrk[f"rbuf{bi}"] for bi in range(NBUF)]
        gsems = [scrk[f"gsem{bi}"] for bi in range(NBUF)]
        ssems = [scrk[f"ssem{bi}"] for bi in range(NBUF)]
        c = lax.axis_index("c")
        s = lax.axis_index("s")

        def grouped(n, issue_load, after_load):
            for g0 in range(0, n, NBUF):
                g = min(NBUF, n - g0)
                lds = [issue_load(g0 + i, i) for i in range(g)]
                sds = []
                for i in range(g):
                    lds[i].wait()
                    sds.extend(after_load(g0 + i, i))
                for sd in sds:
                    sd.wait()

        def load_dim(z_hbm, zA, zB, rpt):
            base = s * rpt

            def ld(j, i):
                sl = pl.ds(base + j * CHUNK, CHUNK)
                return pltpu.async_copy(z_hbm.at[sl], rbufs[i], gsems[i])

            def st(j, i):
                sl = pl.ds(base + j * CHUNK, CHUNK)
                return [pltpu.async_copy(rbufs[i], zA.at[sl], ssems[i]),
                        pltpu.async_copy(rbufs[i], zB.at[sl], gsems[i])]

            grouped(rpt // CHUNK, ld, st)

        def copy_rows(src, dst, rpt):
            base = s * rpt

            def ld(j, i):
                sl = pl.ds(base + j * CHUNK, CHUNK)
                return pltpu.async_copy(src.at[sl], rbufs[i], gsems[i])

            def st(j, i):
                sl = pl.ds(base + j * CHUNK, CHUNK)
                return [pltpu.async_copy(rbufs[i], dst.at[sl], ssems[i])]

            grouped(rpt // CHUNK, ld, st)

        def process_quad(zsrc, zdst, eir):
            gds = [
                pltpu.async_copy(zsrc.at[eir.at[0, bi]], rbufs[bi], gsems[bi])
                for bi in range(NBUF)
            ]
            sds = []
            for bi in range(NBUF):
                gds[bi].wait()
                sds.append(pltpu.async_copy(
                    rbufs[bi], zdst.at[eir.at[1, bi]], ssems[bi], add=True))
            for sd in sds:
                sd.wait()

        def edge_pass(zsrc, zdst, e_hbm, nct):
            base = s * nct
            nq = nct // NBUF
            pltpu.sync_copy(e_hbm.at[0, pl.ds(base, NBUF)], eir0.at[0])
            pltpu.sync_copy(e_hbm.at[1, pl.ds(base, NBUF)], eir0.at[1])

            def pair(h, carry):
                q1 = base + (2 * h + 1) * NBUF
                dB0 = pltpu.async_copy(e_hbm.at[0, pl.ds(q1, NBUF)],
                                       eir1.at[0], isem1)
                dB1 = pltpu.async_copy(e_hbm.at[1, pl.ds(q1, NBUF)],
                                       eir1.at[1], isem1b)
                process_quad(zsrc, zdst, eir0)
                dB0.wait()
                dB1.wait()
                dA0 = pltpu.async_copy(e_hbm.at[0, pl.ds(q1 + NBUF, NBUF)],
                                       eir0.at[0], isem0)
                dA1 = pltpu.async_copy(e_hbm.at[1, pl.ds(q1 + NBUF, NBUF)],
                                       eir0.at[1], isem0b)
                process_quad(zsrc, zdst, eir1)
                dA0.wait()
                dA1.wait()
                return carry

            lax.fori_loop(0, nq // 2, pair, 0)
            if nq % 2 == 1:
                process_quad(zsrc, zdst, eir0)

        def pool_dim(zfin, b_hbm, rpt):
            base = s * rpt

            def ld(j, i):
                sl = pl.ds(base + j * CHUNK, CHUNK)
                pltpu.sync_copy(b_hbm.at[sl], eir0.at[0, i])
                return pltpu.async_copy(zfin.at[sl], rbufs[i], gsems[i])

            def st(j, i):
                return [pltpu.async_copy(rbufs[i], pooled.at[eir0.at[0, i]],
                                         ssems[i], add=True)]

            grouped(rpt // CHUNK, ld, st)

        def run_dims(dims):
            for (zh, zA, zB, eh, bh, nct, rpt) in dims:
                load_dim(zh, zA, zB, rpt)

            @pl.when(s == 0)
            def _():
                zv = jnp.zeros((LANES,), jnp.float32)
                for i in range(BATCH):
                    pstage[i, :] = zv
                pltpu.sync_copy(pstage, pooled)

            plsc.subcore_barrier()

            for layer in range(NUM_LAYERS):
                fwd = layer % 2 == 0
                if layer > 0:
                    for (zh, zA, zB, eh, bh, nct, rpt) in dims:
                        copy_rows(zA if fwd else zB, zB if fwd else zA, rpt)
                    plsc.subcore_barrier()
                for (zh, zA, zB, eh, bh, nct, rpt) in dims:
                    edge_pass(zA if fwd else zB, zB if fwd else zA, eh, nct)
                plsc.subcore_barrier()

            for (zh, zA, zB, eh, bh, nct, rpt) in dims:
                pool_dim(zB if NUM_LAYERS % 2 == 1 else zA, bh, rpt)
            plsc.subcore_barrier()

            @pl.when(s == 0)
            def _():
                pltpu.sync_copy(pooled, pstage)
                pltpu.sync_copy(pstage, out.at[c])

        dims = [(zhs[d], zAs[d], zBs[d], ehs[d], bhs[d], ncts[d], rpts[d])
                for d in range(nd)]

        @pl.when(c == 0)
        def _():
            run_dims(dims[:split])

        @pl.when(c == 1)
        def _():
            run_dims(dims[split:])

    return sc_kernel


def kernel(x0, x1, x2, edge_index0, edge_index1, edge_index2,
           batch0, batch1, batch2, W, b):
    xs = [x0, x1, x2]
    eis = [edge_index0, edge_index1, edge_index2]
    bs = [batch0, batch1, batch2]
    ns = [x.shape[0] for x in xs]
    npads = [_round_up(n + 1, NTILES * CHUNK) for n in ns]
    epads = [_round_up(ei.shape[1], NTILES * CHUNK * NBUF) for ei in eis]

    wp = jnp.pad(W, ((0, 0), (0, LANES - W.shape[1])))
    zs = [_project(xs[i], wp, npads[i]) for i in range(3)]
    es = []
    for i in range(3):
        alloc = epads[i] + NBUF * CHUNK
        pad = alloc - eis[i].shape[1]
        ep = jnp.pad(eis[i], ((0, 0), (0, pad)), constant_values=ns[i])
        es.append(ep.reshape(2, alloc // CHUNK, CHUNK))
    bpads = [jnp.pad(bs[i], (0, npads[i] - ns[i])) for i in range(3)]

    sc = _make_sc_kernel(npads, epads, split=1)
    pooled2 = sc(zs[0], zs[1], zs[2], es[0], es[1], es[2],
                 bpads[0], bpads[1], bpads[2])
    return pooled2.sum(axis=0)[:, : W.shape[1]] + b
